# baseline (speedup 1.0000x reference)
import jax
import jax.numpy as jnp
from jax.experimental import pallas as pl
from jax.experimental.pallas import tpu as pltpu

_R, _C = 2048, 8192
_BR = 128
_G = _R // _BR
_NBUF = 6
_D = 3


def _in_cp(x_hbm, bufs, insems, i):
    return pltpu.make_async_copy(
        x_hbm.at[pl.ds(i * _BR, _BR), :], bufs.at[i % _NBUF], insems.at[i % _NBUF])


def _out_cp(o_hbm, bufs, outsems, i):
    return pltpu.make_async_copy(
        bufs.at[i % _NBUF], o_hbm.at[pl.ds(i * _BR, _BR), :], outsems.at[i % _NBUF])


def _dma_kernel(x_hbm, o_hbm, bufs, insems, outsems):
    for i in range(_D):
        _in_cp(x_hbm, bufs, insems, i).start()
    for i in range(_G):
        _in_cp(x_hbm, bufs, insems, i).wait()
        if i == 0:
            row = jax.lax.broadcasted_iota(jnp.int32, (8, 128), 0)
            col = jax.lax.broadcasted_iota(jnp.int32, (8, 128), 1)
            x = bufs[0, 0:8, 0:128]
            patched = jnp.where(col == 0, 10.0,
                      jnp.where(col == 1, 30.0,
                      jnp.where(col == 2, 20.0,
                      jnp.where(col == 3, 40.0, x))))
            bufs[0, 0:8, 0:128] = jnp.where(row == 0, patched, x)
        _out_cp(o_hbm, bufs, outsems, i).start()
        nxt = i + _D
        if nxt < _G:
            if nxt >= _NBUF:
                _out_cp(o_hbm, bufs, outsems, nxt - _NBUF).wait()
            _in_cp(x_hbm, bufs, insems, nxt).start()
    for i in range(_G - _NBUF, _G):
        _out_cp(o_hbm, bufs, outsems, i).wait()


def kernel(data):
    x = data.reshape(_R, _C)
    out = pl.pallas_call(
        _dma_kernel,
        in_specs=[pl.BlockSpec(memory_space=pl.ANY)],
        out_specs=pl.BlockSpec(memory_space=pl.ANY),
        out_shape=jax.ShapeDtypeStruct((_R, _C), jnp.float32),
        scratch_shapes=[pltpu.VMEM((_NBUF, _BR, _C), jnp.float32),
                        pltpu.SemaphoreType.DMA((_NBUF,)),
                        pltpu.SemaphoreType.DMA((_NBUF,))],
    )(x)
    return out.reshape(-1)


# speedup vs baseline: 2.5175x; 2.5175x over previous
import jax
import jax.numpy as jnp
from jax import lax
from jax.experimental import pallas as pl
from jax.experimental.pallas import tpu as pltpu
from jax.experimental.pallas import tpu_sc as plsc

_N = 16777216
_NC, _NS = 2, 16
_NW = _NC * _NS
_SHARD = _N // _NW
_CHUNK = 65536
_NCHUNK = _SHARD // _CHUNK


def _sc_body(x_hbm, o_hbm, buf, buf16):
    wid = lax.axis_index("s") * _NC + lax.axis_index("c")
    base = wid * _SHARD
    for c in range(_NCHUNK):
        off = base + c * _CHUNK
        pltpu.sync_copy(x_hbm.at[pl.ds(off, _CHUNK)], buf)
        pltpu.sync_copy(buf, o_hbm.at[pl.ds(off, _CHUNK)])

    @pl.when(wid == 0)
    def _patch():
        pltpu.sync_copy(x_hbm.at[pl.ds(0, 16)], buf16)
        i = lax.iota(jnp.int32, 16)
        v = buf16[...]
        buf16[...] = jnp.where(i == 0, 10.0,
                     jnp.where(i == 1, 30.0,
                     jnp.where(i == 2, 20.0,
                     jnp.where(i == 3, 40.0, v))))
        pltpu.sync_copy(buf16, o_hbm.at[pl.ds(0, 16)])


def kernel(data):
    mesh = plsc.VectorSubcoreMesh(core_axis_name="c", subcore_axis_name="s")
    f = pl.kernel(
        _sc_body,
        out_type=jax.ShapeDtypeStruct((_N,), jnp.float32),
        mesh=mesh,
        scratch_types=[pltpu.VMEM((_CHUNK,), jnp.float32),
                       pltpu.VMEM((16,), jnp.float32)],
    )
    return f(data)


# speedup vs baseline: 2.5995x; 1.0326x over previous
import jax
import jax.numpy as jnp
from jax import lax
from jax.experimental import pallas as pl
from jax.experimental.pallas import tpu as pltpu
from jax.experimental.pallas import tpu_sc as plsc

_N = 16777216
_NC, _NS = 2, 16
_NW = _NC * _NS
_SHARD = _N // _NW
_CHUNK = 32768
_NCHUNK = _SHARD // _CHUNK
_NBUF = 3
_D = 2


def _sc_body(x_hbm, o_hbm, b0, b1, b2, buf16, si0, si1, si2, so0, so1, so2):
    bufs = (b0, b1, b2)
    insems = (si0, si1, si2)
    outsems = (so0, so1, so2)
    wid = lax.axis_index("s") * _NC + lax.axis_index("c")
    base = wid * _SHARD

    def in_cp(c):
        return pltpu.make_async_copy(
            x_hbm.at[pl.ds(base + c * _CHUNK, _CHUNK)],
            bufs[c % _NBUF], insems[c % _NBUF])

    def out_cp(c):
        return pltpu.make_async_copy(
            bufs[c % _NBUF],
            o_hbm.at[pl.ds(base + c * _CHUNK, _CHUNK)], outsems[c % _NBUF])

    for c in range(_D):
        in_cp(c).start()
    for c in range(_NCHUNK):
        in_cp(c).wait()
        out_cp(c).start()
        nxt = c + _D
        if nxt < _NCHUNK:
            if nxt >= _NBUF:
                out_cp(nxt - _NBUF).wait()
            in_cp(nxt).start()
    for c in range(_NCHUNK - _NBUF, _NCHUNK):
        out_cp(c).wait()

    @pl.when(wid == 0)
    def _patch():
        pltpu.sync_copy(x_hbm.at[pl.ds(0, 16)], buf16)
        i = lax.iota(jnp.int32, 16)
        v = buf16[...]
        buf16[...] = jnp.where(i == 0, 10.0,
                     jnp.where(i == 1, 30.0,
                     jnp.where(i == 2, 20.0,
                     jnp.where(i == 3, 40.0, v))))
        pltpu.sync_copy(buf16, o_hbm.at[pl.ds(0, 16)])


def kernel(data):
    mesh = plsc.VectorSubcoreMesh(core_axis_name="c", subcore_axis_name="s")
    f = pl.kernel(
        _sc_body,
        out_type=jax.ShapeDtypeStruct((_N,), jnp.float32),
        mesh=mesh,
        scratch_types=[pltpu.VMEM((_CHUNK,), jnp.float32)] * _NBUF
                      + [pltpu.VMEM((16,), jnp.float32)]
                      + [pltpu.SemaphoreType.DMA] * (2 * _NBUF),
    )
    return f(data)
